# Initial kernel scaffold; baseline (speedup 1.0000x reference)
import jax
import jax.numpy as jnp
from jax.experimental import pallas as pl


def kernel(indices, weight):
    raise NotImplementedError("write your pallas kernel here")


# speedup vs baseline: 4.8018x; 4.8018x over previous
import functools

import jax
import jax.numpy as jnp
from jax import lax
from jax.experimental import pallas as pl
from jax.experimental.pallas import tpu as pltpu
from jax.experimental.pallas import tpu_sc as plsc

NUM_CORES = 2
NUM_SUBCORES = 16
NUM_WORKERS = NUM_CORES * NUM_SUBCORES

GATHER = 128
CHUNK_GATHERS = 8
CHUNK = GATHER * CHUNK_GATHERS


@functools.partial(jax.jit, static_argnums=())
def _sc_embedding_lookup(table, idx2d):
    n_gather_rows, g = idx2d.shape
    assert g == GATHER
    B = n_gather_rows * GATHER
    D = table.shape[1]
    rows_per_w = B // NUM_WORKERS
    chunks_per_w = rows_per_w // CHUNK
    assert rows_per_w % CHUNK == 0

    mesh = plsc.VectorSubcoreMesh(core_axis_name="c", subcore_axis_name="s")

    @functools.partial(
        pl.kernel,
        out_type=jax.ShapeDtypeStruct((B, D), jnp.float32),
        mesh=mesh,
        compiler_params=pltpu.CompilerParams(use_tc_tiling_on_sc=False),
        scratch_types=[
            pltpu.VMEM((CHUNK_GATHERS, GATHER), jnp.int32),
            pltpu.VMEM((CHUNK, D), jnp.float32),
            pltpu.SemaphoreType.DMA,
        ],
    )
    def k(table_hbm, idx_hbm, out_hbm, idx_v, rows_v, sem):
        wid = lax.axis_index("s") * NUM_CORES + lax.axis_index("c")
        row0 = wid * rows_per_w
        g0 = row0 // GATHER

        def body(ci, carry):
            base = pl.multiple_of(row0 + ci * CHUNK, CHUNK)
            gbase = pl.multiple_of(g0 + ci * CHUNK_GATHERS, CHUNK_GATHERS)
            pltpu.sync_copy(idx_hbm.at[pl.ds(gbase, CHUNK_GATHERS)], idx_v)
            copies = []
            for j in range(CHUNK_GATHERS):
                copies.append(
                    pltpu.async_copy(
                        table_hbm.at[idx_v.at[j]],
                        rows_v.at[pl.ds(j * GATHER, GATHER)],
                        sem,
                    )
                )
            for c in copies:
                c.wait()
            pltpu.sync_copy(rows_v, out_hbm.at[pl.ds(base, CHUNK)])
            return carry

        lax.fori_loop(0, chunks_per_w, body, 0)

    return k(table, idx2d)


def kernel(indices, weight):
    B = indices.shape[0] * indices.shape[1]
    idx2d = indices.reshape(B // GATHER, GATHER).astype(jnp.int32)
    out = _sc_embedding_lookup(weight, idx2d)
    return out.reshape(indices.shape + (weight.shape[1],))


# speedup vs baseline: 4.9783x; 1.0367x over previous
import functools

import jax
import jax.numpy as jnp
from jax import lax
from jax.experimental import pallas as pl
from jax.experimental.pallas import tpu as pltpu
from jax.experimental.pallas import tpu_sc as plsc

NUM_CORES = 2
NUM_SUBCORES = 16
NUM_WORKERS = NUM_CORES * NUM_SUBCORES

GATHER = 128
CHUNK_GATHERS = 8
CHUNK = GATHER * CHUNK_GATHERS


def _sc_embedding_lookup(table, idx2d):
    n_gather_rows, g = idx2d.shape
    assert g == GATHER
    B = n_gather_rows * GATHER
    D = table.shape[1]
    rows_per_w = B // NUM_WORKERS
    chunks_per_w = rows_per_w // CHUNK
    assert rows_per_w % CHUNK == 0 and chunks_per_w % 2 == 0
    npairs = chunks_per_w // 2

    mesh = plsc.VectorSubcoreMesh(core_axis_name="c", subcore_axis_name="s")

    @functools.partial(
        pl.kernel,
        out_type=jax.ShapeDtypeStruct((B, D), jnp.float32),
        mesh=mesh,
        compiler_params=pltpu.CompilerParams(use_tc_tiling_on_sc=False),
        scratch_types=[
            pltpu.VMEM((2, CHUNK_GATHERS, GATHER), jnp.int32),
            pltpu.VMEM((2, CHUNK, D), jnp.float32),
            pltpu.SemaphoreType.DMA,
            pltpu.SemaphoreType.DMA,
            pltpu.SemaphoreType.DMA,
            pltpu.SemaphoreType.DMA,
        ],
    )
    def k(table_hbm, idx_hbm, out_hbm, idx_v, rows_v, gsem0, gsem1, ssem0, ssem1):
        wid = lax.axis_index("s") * NUM_CORES + lax.axis_index("c")
        row0 = wid * rows_per_w
        g0 = row0 // GATHER

        def idx_load(c, slot):
            gbase = pl.multiple_of(g0 + c * CHUNK_GATHERS, CHUNK_GATHERS)
            pltpu.sync_copy(idx_hbm.at[pl.ds(gbase, CHUNK_GATHERS)], idx_v.at[slot])

        def fire_gathers(c, slot, sem):
            del c
            for j in range(CHUNK_GATHERS):
                pltpu.async_copy(
                    table_hbm.at[idx_v.at[slot, j]],
                    rows_v.at[slot, pl.ds(j * GATHER, GATHER)],
                    sem,
                )

        def drain_gathers(slot, sem):
            pltpu.make_async_copy(
                table_hbm.at[pl.ds(0, CHUNK)], rows_v.at[slot], sem
            ).wait()

        def fire_store(c, slot, sem):
            base = pl.multiple_of(row0 + c * CHUNK, CHUNK)
            pltpu.async_copy(rows_v.at[slot], out_hbm.at[pl.ds(base, CHUNK)], sem)

        def drain_store(slot, sem):
            pltpu.make_async_copy(
                rows_v.at[slot], out_hbm.at[pl.ds(0, CHUNK)], sem
            ).wait()

        idx_load(0, 0)
        fire_gathers(0, 0, gsem0)

        def pair(gi, carry):
            c0 = 2 * gi
            idx_load(c0 + 1, 1)
            drain_gathers(0, gsem0)

            @pl.when(gi >= 1)
            def _():
                drain_store(1, ssem1)

            fire_gathers(c0 + 1, 1, gsem1)
            fire_store(c0, 0, ssem0)

            drain_gathers(1, gsem1)
            drain_store(0, ssem0)

            @pl.when(gi < npairs - 1)
            def _():
                idx_load(c0 + 2, 0)
                fire_gathers(c0 + 2, 0, gsem0)

            fire_store(c0 + 1, 1, ssem1)
            return carry

        lax.fori_loop(0, npairs, pair, 0)
        drain_store(1, ssem1)

    return k(table, idx2d)


def kernel(indices, weight):
    B = indices.shape[0] * indices.shape[1]
    idx2d = indices.reshape(B // GATHER, GATHER).astype(jnp.int32)
    out = _sc_embedding_lookup(weight, idx2d)
    return out.reshape(indices.shape + (weight.shape[1],))
